# baseline (speedup 1.0000x reference)
import functools

import jax
import jax.numpy as jnp
from jax import lax
from jax.experimental import pallas as pl
from jax.experimental.pallas import tpu as pltpu
from jax.experimental.pallas import tpu_sc as plsc

NUM_CORES = 2
NUM_SUBCORES = 16
NW = NUM_CORES * NUM_SUBCORES
LANES = 16

G = 128


def _sc_body(idx_hbm, table_hbm, out_hbm, idx_v, rows_v, out_v, sidx_v,
             base_v, gsem0, gsem1, osem0, osem1,
             *, blocks_per_w, blocks_per_l, l_len, d, stage_rows):
    nsl = d // LANES
    wid = lax.axis_index("s") * NUM_CORES + lax.axis_index("c")
    blk0 = wid * blocks_per_w
    gsems = (gsem0, gsem1)
    osems = (osem0, osem1)

    row_lo = wid * (2 * blocks_per_w)
    astart = (row_lo // 8) * 8
    off = row_lo - astart
    pltpu.sync_copy(
        idx_hbm.at[pl.ds(pl.multiple_of(astart, 8), stage_rows)], idx_v)

    for s in range(G // LANES):
        base_v[pl.ds(s * LANES, LANES)] = (
            lax.iota(jnp.int32, LANES) + s * LANES) * l_len

    def start_g(j, b):
        pltpu.async_copy(table_hbm.at[idx_v.at[off + 2 * j]],
                         rows_v.at[b, pl.ds(0, G)], gsems[b])
        pltpu.async_copy(table_hbm.at[idx_v.at[off + 2 * j + 1]],
                         rows_v.at[b, pl.ds(G, G)], gsems[b])

    def wait_g(b):
        pltpu.make_async_copy(table_hbm.at[pl.ds(0, 2 * G)], rows_v.at[b],
                              gsems[b]).wait()

    def wait_o(b):
        pltpu.make_async_copy(out_v.at[b], out_hbm.at[sidx_v.at[b]],
                              osems[b]).wait()

    def compute_store(j, b, first):
        if not first:
            wait_o(b)
        blk = blk0 + j
        l = blk // blocks_per_l
        tc = blk - l * blocks_per_l
        cst = tc * (G * l_len) + l
        for s in range(G // LANES):
            sl = pl.ds(s * LANES, LANES)
            sidx_v[b, sl] = base_v[sl] + cst
        rref = rows_v.at[b]
        oref = out_v.at[b]

        @plsc.parallel_loop(0, G, unroll=8)
        def compute_row(c):
            for s in range(nsl):
                sl = pl.ds(s * LANES, LANES)
                oref[c, sl] = rref[c, sl] + rref[G + c, sl]

        pltpu.async_copy(oref, out_hbm.at[sidx_v.at[b]], osems[b])

    start_g(0, 0)
    start_g(1, 1)
    wait_g(0)
    compute_store(0, 0, True)
    start_g(2, 0)
    wait_g(1)
    compute_store(1, 1, True)

    def outer(kk, carry):
        j = 2 * kk
        start_g(j + 1, 1)
        wait_g(0)
        compute_store(j, 0, False)
        start_g(j + 2, 0)
        wait_g(1)
        compute_store(j + 1, 1, False)
        return carry

    lax.fori_loop(1, blocks_per_w // 2 - 1, outer, 0)
    start_g(blocks_per_w - 1, 1)
    wait_g(0)
    compute_store(blocks_per_w - 2, 0, False)
    wait_g(1)
    compute_store(blocks_per_w - 1, 1, False)
    wait_o(0)
    wait_o(1)


@functools.partial(jax.jit, static_argnums=(2, 3, 4))
def _sc_embed(idx2d, we, n_rows, l_len, d):
    blocks = (2 * n_rows) // (2 * G)
    blocks_per_w = blocks // NW
    blocks_per_l = n_rows // (l_len * G)
    assert blocks_per_w % 2 == 0 and blocks_per_w >= 6
    max_off = max((2 * blocks_per_w * w) % 8 for w in range(NW))
    stage_rows = 2 * blocks_per_w + max_off
    assert ((2 * blocks_per_w * (NW - 1)) // 8) * 8 + stage_rows \
        <= idx2d.shape[0]
    body = functools.partial(_sc_body, blocks_per_w=blocks_per_w,
                             blocks_per_l=blocks_per_l, l_len=l_len, d=d,
                             stage_rows=stage_rows)
    k = pl.kernel(
        body,
        out_type=jax.ShapeDtypeStruct((n_rows, d), jnp.float32),
        mesh=plsc.VectorSubcoreMesh(core_axis_name="c", subcore_axis_name="s"),
        scratch_types=[
            pltpu.VMEM((stage_rows, G), jnp.int32),
            pltpu.VMEM((2, 2 * G, d), jnp.float32),
            pltpu.VMEM((2, G, d), jnp.float32),
            pltpu.VMEM((2, G), jnp.int32),
            pltpu.VMEM((G,), jnp.int32),
            pltpu.SemaphoreType.DMA,
            pltpu.SemaphoreType.DMA,
            pltpu.SemaphoreType.DMA,
            pltpu.SemaphoreType.DMA,
        ],
    )
    return k(idx2d, we)


def kernel(inputs, we):
    b, l_len, two = inputs.shape
    n_rows = b * l_len
    d = we.shape[1]
    blocks_per_l = b // G
    x8 = inputs.astype(jnp.int32).reshape(blocks_per_l, G, l_len, two)
    wphys = jnp.transpose(x8, (2, 0, 3, 1))
    idx2d = wphys.reshape((n_rows * two) // G, G)
    out = _sc_embed(idx2d, we, n_rows, l_len, d)
    return out.reshape(b, l_len, d)


# speedup vs baseline: 1.0098x; 1.0098x over previous
import functools

import jax
import jax.numpy as jnp
from jax import lax
from jax.experimental import pallas as pl
from jax.experimental.pallas import tpu as pltpu
from jax.experimental.pallas import tpu_sc as plsc

NUM_CORES = 2
NUM_SUBCORES = 16
NW = NUM_CORES * NUM_SUBCORES
LANES = 16

G = 128


def _sc_body(idx_hbm, table_hbm, out_hbm, idx_v, rows_v, out_v, sidx_v,
             base_v, gsem0, gsem1, osem0, osem1,
             *, blocks_per_w, blocks_per_l, l_len, d, stage_rows):
    nsl = d // LANES
    wid = lax.axis_index("s") * NUM_CORES + lax.axis_index("c")
    blk0 = wid * blocks_per_w
    gsems = (gsem0, gsem1)
    osems = (osem0, osem1)

    row_lo = wid * (2 * blocks_per_w)
    astart = (row_lo // 8) * 8
    off = row_lo - astart
    pltpu.sync_copy(
        idx_hbm.at[pl.ds(pl.multiple_of(astart, 8), stage_rows)], idx_v)

    for s in range(G // LANES):
        base_v[pl.ds(s * LANES, LANES)] = (
            lax.iota(jnp.int32, LANES) + s * LANES) * l_len

    def start_g(j, b):
        pltpu.async_copy(table_hbm.at[idx_v.at[off + 2 * j]],
                         rows_v.at[b, pl.ds(0, G)], gsems[b])
        pltpu.async_copy(table_hbm.at[idx_v.at[off + 2 * j + 1]],
                         rows_v.at[b, pl.ds(G, G)], gsems[b])

    def wait_g(b):
        pltpu.make_async_copy(table_hbm.at[pl.ds(0, 2 * G)], rows_v.at[b],
                              gsems[b]).wait()

    def wait_o(b):
        pltpu.make_async_copy(out_v.at[b], out_hbm.at[sidx_v.at[b]],
                              osems[b]).wait()

    def compute_store(j, b, first):
        if not first:
            wait_o(b)
        blk = blk0 + j
        l = blk // blocks_per_l
        tc = blk - l * blocks_per_l
        cst = tc * (G * l_len) + l
        for s in range(G // LANES):
            sl = pl.ds(s * LANES, LANES)
            sidx_v[b, sl] = base_v[sl] + cst
        rref = rows_v.at[b]
        oref = out_v.at[b]

        @plsc.parallel_loop(0, G, unroll=4)
        def compute_row(c):
            for s in range(nsl):
                sl = pl.ds(s * LANES, LANES)
                oref[c, sl] = rref[c, sl] + rref[G + c, sl]

        pltpu.async_copy(oref, out_hbm.at[sidx_v.at[b]], osems[b])

    start_g(0, 0)
    start_g(1, 1)
    wait_g(0)
    compute_store(0, 0, True)
    start_g(2, 0)
    wait_g(1)
    compute_store(1, 1, True)

    def outer(kk, carry):
        j = 2 * kk
        start_g(j + 1, 1)
        wait_g(0)
        compute_store(j, 0, False)
        start_g(j + 2, 0)
        wait_g(1)
        compute_store(j + 1, 1, False)
        return carry

    lax.fori_loop(1, blocks_per_w // 2 - 1, outer, 0)
    start_g(blocks_per_w - 1, 1)
    wait_g(0)
    compute_store(blocks_per_w - 2, 0, False)
    wait_g(1)
    compute_store(blocks_per_w - 1, 1, False)
    wait_o(0)
    wait_o(1)


@functools.partial(jax.jit, static_argnums=(2, 3, 4))
def _sc_embed(idx2d, we, n_rows, l_len, d):
    blocks = (2 * n_rows) // (2 * G)
    blocks_per_w = blocks // NW
    blocks_per_l = n_rows // (l_len * G)
    assert blocks_per_w % 2 == 0 and blocks_per_w >= 6
    max_off = max((2 * blocks_per_w * w) % 8 for w in range(NW))
    stage_rows = 2 * blocks_per_w + max_off
    assert ((2 * blocks_per_w * (NW - 1)) // 8) * 8 + stage_rows \
        <= idx2d.shape[0]
    body = functools.partial(_sc_body, blocks_per_w=blocks_per_w,
                             blocks_per_l=blocks_per_l, l_len=l_len, d=d,
                             stage_rows=stage_rows)
    k = pl.kernel(
        body,
        out_type=jax.ShapeDtypeStruct((n_rows, d), jnp.float32),
        mesh=plsc.VectorSubcoreMesh(core_axis_name="c", subcore_axis_name="s"),
        scratch_types=[
            pltpu.VMEM((stage_rows, G), jnp.int32),
            pltpu.VMEM((2, 2 * G, d), jnp.float32),
            pltpu.VMEM((2, G, d), jnp.float32),
            pltpu.VMEM((2, G), jnp.int32),
            pltpu.VMEM((G,), jnp.int32),
            pltpu.SemaphoreType.DMA,
            pltpu.SemaphoreType.DMA,
            pltpu.SemaphoreType.DMA,
            pltpu.SemaphoreType.DMA,
        ],
    )
    return k(idx2d, we)


def kernel(inputs, we):
    b, l_len, two = inputs.shape
    n_rows = b * l_len
    d = we.shape[1]
    blocks_per_l = b // G
    x8 = inputs.astype(jnp.int32).reshape(blocks_per_l, G, l_len, two)
    wphys = jnp.transpose(x8, (2, 0, 3, 1))
    idx2d = wphys.reshape((n_rows * two) // G, G)
    out = _sc_embed(idx2d, we, n_rows, l_len, d)
    return out.reshape(b, l_len, d)
